# Initial kernel scaffold; baseline (speedup 1.0000x reference)
import jax
import jax.numpy as jnp
from jax.experimental import pallas as pl


def kernel(logits, attention_mask):
    raise NotImplementedError("write your pallas kernel here")


# speedup vs baseline: 1.9308x; 1.9308x over previous
import jax
import jax.numpy as jnp
from jax.experimental import pallas as pl

INF_ = 1e12


def _tc_body(maskc_ref, maskr_ref, logits_ref, x_ref, pos_ref):
    S = logits_ref.shape[1]
    amc = maskc_ref[...]
    amr = maskr_ref[...]
    logits = logits_ref[...]
    pair = (amr * amc) != 0
    x = jnp.where(pair, logits, -INF_)
    row = jax.lax.broadcasted_iota(jnp.int32, (1, S, S), 1)
    col = jax.lax.broadcasted_iota(jnp.int32, (1, S, S), 2)
    row_adj = jnp.where((row == 0) | (row == S - 1), -INF_, 0.0)
    col_adj = jnp.where((col == 0) | (col == S - 1), -INF_, 0.0)
    x = x + row_adj
    x = x + col_adj
    x_ref[...] = x
    pos_ref[...] = x > 0


def kernel(logits, attention_mask):
    B, L, S, _ = logits.shape
    flat = logits.reshape(B * L, S, S)
    am3 = attention_mask.reshape(B, 1, S)
    amt = attention_mask.reshape(B, S, 1)
    grid = (B * L,)
    x, pos = pl.pallas_call(
        _tc_body,
        grid=grid,
        in_specs=[
            pl.BlockSpec((1, 1, S), lambda r: (r // L, 0, 0)),
            pl.BlockSpec((1, S, 1), lambda r: (r // L, 0, 0)),
            pl.BlockSpec((1, S, S), lambda r: (r, 0, 0)),
        ],
        out_specs=[
            pl.BlockSpec((1, S, S), lambda r: (r, 0, 0)),
            pl.BlockSpec((1, S, S), lambda r: (r, 0, 0)),
        ],
        out_shape=[
            jax.ShapeDtypeStruct((B * L, S, S), jnp.float32),
            jax.ShapeDtypeStruct((B * L, S, S), jnp.bool_),
        ],
    )(am3, amt, flat)
    return x.reshape(B, L, S, S), pos.reshape(B, L, S, S)
